# Initial kernel scaffold; baseline (speedup 1.0000x reference)
import jax
import jax.numpy as jnp
from jax.experimental import pallas as pl


def kernel(x, edge_index, W1_rel, W1_root, b1, g1, be1, W2_rel, W2_root, b2, g2, be2, W3_rel, W3_root, b3, lin_W, lin_b):
    raise NotImplementedError("write your pallas kernel here")


# speedup vs baseline: 7.1173x; 7.1173x over previous
import functools

import jax
import jax.numpy as jnp
from jax import lax
from jax.experimental import pallas as pl
from jax.experimental.pallas import tpu as pltpu
from jax.experimental.pallas import tpu_sc as plsc

N = 10000
E = 320000
F = 128
H = 20
L_OUT = 10

HP = 32
N_PAD = 10112
TRASH = N

NC = 2
NS = 16
NW = NC * NS
EB = 128
NB_W = 80
NB_TOT = NW * NB_W
E_PAD = NB_TOT * EB
ROWS_T = N_PAD // NS


def _sc_scatter_body(p_hbm, src_hbm, dst_hbm, zero_hbm, out_hbm,
                     src_v, dst_v, rows_v, acc, sem):
    c = lax.axis_index("c")
    s = lax.axis_index("s")
    wid = c * NS + s
    row0 = pl.multiple_of(s * ROWS_T, 8)

    pltpu.sync_copy(zero_hbm.at[pl.ds(row0, ROWS_T)],
                    acc.at[pl.ds(row0, ROWS_T)])
    pltpu.sync_copy(src_hbm.at[pl.ds(wid * NB_W, NB_W)], src_v)
    pltpu.sync_copy(dst_hbm.at[pl.ds(wid * NB_W, NB_W)], dst_v)
    plsc.subcore_barrier()

    def step(j, carry):
        pltpu.async_copy(p_hbm.at[src_v.at[j]], rows_v, sem).wait()
        pltpu.sync_copy(rows_v, acc.at[dst_v.at[j]], add=True)
        return carry

    lax.fori_loop(0, NB_W, step, 0)
    plsc.subcore_barrier()
    pltpu.sync_copy(acc.at[pl.ds(row0, ROWS_T)],
                    out_hbm.at[c].at[pl.ds(row0, ROWS_T)])


@functools.cache
def _sc_scatter():
    return pl.kernel(
        _sc_scatter_body,
        out_type=jax.ShapeDtypeStruct((NC, N_PAD, HP), jnp.float32),
        mesh=plsc.VectorSubcoreMesh(core_axis_name="c", subcore_axis_name="s",
                                    num_cores=NC, num_subcores=NS),
        compiler_params=pltpu.CompilerParams(use_tc_tiling_on_sc=False),
        scratch_types=[
            pltpu.VMEM((NB_W, EB), jnp.int32),
            pltpu.VMEM((NB_W, EB), jnp.int32),
            pltpu.VMEM((EB, HP), jnp.float32),
            pltpu.VMEM_SHARED((N_PAD, HP), jnp.float32),
            pltpu.SemaphoreType.DMA,
        ],
    )


def _dot_t(a, b):
    return lax.dot_general(a, b, (((1,), (1,)), ((), ())),
                           precision=lax.Precision.HIGHEST,
                           preferred_element_type=jnp.float32)


def _tc_pre_body(x_ref, wrel_ref, wroot_ref, p_ref, r_ref):
    x = x_ref[...]
    p_ref[...] = _dot_t(x, wrel_ref[...])
    r_ref[...] = _dot_t(x, wroot_ref[...])


def _tc_mid_body(acc_ref, r_ref, b_ref, g_ref, be_ref, wrel_ref, wroot_ref,
                 h_ref, p_ref, rn_ref):
    s = acc_ref[0] + acc_ref[1] + r_ref[...] + b_ref[...]
    t = jnp.maximum(s, 0.0)
    tv = t[:N]
    mu = jnp.mean(tv, axis=0, keepdims=True)
    var = jnp.mean(tv * tv, axis=0, keepdims=True) - mu * mu
    h = (t - mu) * lax.rsqrt(var + 1e-5) * g_ref[...] + be_ref[...]
    h_ref[...] = h
    p_ref[...] = _dot_t(h, wrel_ref[...])
    rn_ref[...] = _dot_t(h, wroot_ref[...])


def _tc_post_body(acc_ref, r_ref, b_ref, h1_ref, h2_ref, lw_ref, lb_ref,
                  out_ref):
    s = acc_ref[0] + acc_ref[1] + r_ref[...] + b_ref[...]
    h3 = jnp.maximum(s, 0.0)
    cat = jnp.concatenate([h1_ref[...], h2_ref[...], h3], axis=1)
    y = _dot_t(cat, lw_ref[...]) + lb_ref[...]
    out_ref[...] = y[:N, :L_OUT]


_f32 = lambda *shape: jax.ShapeDtypeStruct(shape, jnp.float32)

_tc_pre = pl.pallas_call(
    _tc_pre_body, out_shape=(_f32(N_PAD, HP), _f32(N_PAD, HP)))

_tc_mid = pl.pallas_call(
    _tc_mid_body,
    out_shape=(_f32(N_PAD, HP), _f32(N_PAD, HP), _f32(N_PAD, HP)))

_tc_post = pl.pallas_call(_tc_post_body, out_shape=_f32(N, L_OUT))


def _pad_w(w, rows, cols):
    return jnp.pad(w, ((0, rows - w.shape[0]), (0, cols - w.shape[1])))


def _pad_v(v):
    return jnp.pad(v, (0, HP - v.shape[0]))[None, :]


def kernel(x, edge_index, W1_rel, W1_root, b1, g1, be1,
           W2_rel, W2_root, b2, g2, be2,
           W3_rel, W3_root, b3, lin_W, lin_b):
    x_pad = jnp.pad(x, ((0, N_PAD - N), (0, 0)))

    npad = E_PAD - E
    src2 = jnp.concatenate(
        [edge_index[0], jnp.zeros((npad,), jnp.int32)]).reshape(NB_TOT, EB)
    dst2 = jnp.concatenate(
        [edge_index[1], jnp.full((npad,), TRASH, jnp.int32)]).reshape(NB_TOT, EB)

    zeros = jnp.zeros((N_PAD, HP), jnp.float32)

    w1r = _pad_w(W1_rel, HP, F)
    w1o = _pad_w(W1_root, HP, F)
    w2r = _pad_w(W2_rel, HP, HP)
    w2o = _pad_w(W2_root, HP, HP)
    w3r = _pad_w(W3_rel, HP, HP)
    w3o = _pad_w(W3_root, HP, HP)
    lw = jnp.zeros((16, 3 * HP), jnp.float32)
    for i in range(3):
        lw = lw.at[:L_OUT, i * HP:i * HP + H].set(lin_W[:, i * H:(i + 1) * H])
    lb = jnp.pad(lin_b, (0, 16 - L_OUT))[None, :]

    sc = _sc_scatter()
    p1, r1 = _tc_pre(x_pad, w1r, w1o)
    acc1 = sc(p1, src2, dst2, zeros)
    h1, p2, r2 = _tc_mid(acc1, r1, _pad_v(b1), _pad_v(g1), _pad_v(be1),
                         w2r, w2o)
    acc2 = sc(p2, src2, dst2, zeros)
    h2, p3, r3 = _tc_mid(acc2, r2, _pad_v(b2), _pad_v(g2), _pad_v(be2),
                         w3r, w3o)
    acc3 = sc(p3, src2, dst2, zeros)
    out = _tc_post(acc3, r3, _pad_v(b3), h1, h2, lw, lb)
    return out


# speedup vs baseline: 8.9582x; 1.2586x over previous
import functools

import jax
import jax.numpy as jnp
from jax import lax
from jax.experimental import pallas as pl
from jax.experimental.pallas import tpu as pltpu
from jax.experimental.pallas import tpu_sc as plsc

N = 10000
E = 320000
F = 128
H = 20
L_OUT = 10

HP = 32
N_PAD = 10112
TRASH = N

NC = 2
NS = 16
NW = NC * NS
EB = 128
NB_W = 80
NB_TOT = NW * NB_W
E_PAD = NB_TOT * EB
ROWS_T = N_PAD // NS


def _sc_scatter_body(p_hbm, src_hbm, dst_hbm, zero_hbm, out_hbm,
                     src_v, dst_v, rows0, rows1, acc, sem0, sem1):
    c = lax.axis_index("c")
    s = lax.axis_index("s")
    wid = c * NS + s
    row0 = pl.multiple_of(s * ROWS_T, 8)

    pltpu.sync_copy(zero_hbm.at[pl.ds(row0, ROWS_T)],
                    acc.at[pl.ds(row0, ROWS_T)])
    pltpu.sync_copy(src_hbm.at[pl.ds(wid * NB_W, NB_W)], src_v)
    pltpu.sync_copy(dst_hbm.at[pl.ds(wid * NB_W, NB_W)], dst_v)
    plsc.subcore_barrier()

    pltpu.async_copy(p_hbm.at[src_v.at[0]], rows0, sem0)

    def step(t, carry):
        j0 = 2 * t
        j1 = 2 * t + 1
        j2 = jnp.minimum(2 * t + 2, NB_W - 1)
        pltpu.async_copy(p_hbm.at[src_v.at[j1]], rows1, sem1)
        pltpu.make_async_copy(p_hbm.at[src_v.at[j0]], rows0, sem0).wait()
        pltpu.sync_copy(rows0, acc.at[dst_v.at[j0]], add=True)
        pltpu.async_copy(p_hbm.at[src_v.at[j2]], rows0, sem0)
        pltpu.make_async_copy(p_hbm.at[src_v.at[j1]], rows1, sem1).wait()
        pltpu.sync_copy(rows1, acc.at[dst_v.at[j1]], add=True)
        return carry

    lax.fori_loop(0, NB_W // 2, step, 0)
    pltpu.make_async_copy(p_hbm.at[src_v.at[NB_W - 1]], rows0, sem0).wait()
    plsc.subcore_barrier()
    pltpu.sync_copy(acc.at[pl.ds(row0, ROWS_T)],
                    out_hbm.at[c].at[pl.ds(row0, ROWS_T)])


@functools.cache
def _sc_scatter():
    return pl.kernel(
        _sc_scatter_body,
        out_type=jax.ShapeDtypeStruct((NC, N_PAD, HP), jnp.float32),
        mesh=plsc.VectorSubcoreMesh(core_axis_name="c", subcore_axis_name="s",
                                    num_cores=NC, num_subcores=NS),
        compiler_params=pltpu.CompilerParams(use_tc_tiling_on_sc=False),
        scratch_types=[
            pltpu.VMEM((NB_W, EB), jnp.int32),
            pltpu.VMEM((NB_W, EB), jnp.int32),
            pltpu.VMEM((EB, HP), jnp.float32),
            pltpu.VMEM((EB, HP), jnp.float32),
            pltpu.VMEM_SHARED((N_PAD, HP), jnp.float32),
            pltpu.SemaphoreType.DMA,
            pltpu.SemaphoreType.DMA,
        ],
    )


def _dot_t(a, b):
    return lax.dot_general(a, b, (((1,), (1,)), ((), ())),
                           precision=lax.Precision.HIGHEST,
                           preferred_element_type=jnp.float32)


def _tc_pre_body(x_ref, wrel_ref, wroot_ref, p_ref, r_ref):
    x = x_ref[...]
    p_ref[...] = _dot_t(x, wrel_ref[...])
    r_ref[...] = _dot_t(x, wroot_ref[...])


def _tc_mid_body(acc_ref, r_ref, b_ref, g_ref, be_ref, wrel_ref, wroot_ref,
                 h_ref, p_ref, rn_ref):
    s = acc_ref[0] + acc_ref[1] + r_ref[...] + b_ref[...]
    t = jnp.maximum(s, 0.0)
    tv = t[:N]
    mu = jnp.mean(tv, axis=0, keepdims=True)
    var = jnp.mean(tv * tv, axis=0, keepdims=True) - mu * mu
    h = (t - mu) * lax.rsqrt(var + 1e-5) * g_ref[...] + be_ref[...]
    h_ref[...] = h
    p_ref[...] = _dot_t(h, wrel_ref[...])
    rn_ref[...] = _dot_t(h, wroot_ref[...])


def _tc_post_body(acc_ref, r_ref, b_ref, h1_ref, h2_ref, lw_ref, lb_ref,
                  out_ref):
    s = acc_ref[0] + acc_ref[1] + r_ref[...] + b_ref[...]
    h3 = jnp.maximum(s, 0.0)
    cat = jnp.concatenate([h1_ref[...], h2_ref[...], h3], axis=1)
    y = _dot_t(cat, lw_ref[...]) + lb_ref[...]
    out_ref[...] = y[:N, :L_OUT]


_f32 = lambda *shape: jax.ShapeDtypeStruct(shape, jnp.float32)

_tc_pre = pl.pallas_call(
    _tc_pre_body, out_shape=(_f32(N_PAD, HP), _f32(N_PAD, HP)))

_tc_mid = pl.pallas_call(
    _tc_mid_body,
    out_shape=(_f32(N_PAD, HP), _f32(N_PAD, HP), _f32(N_PAD, HP)))

_tc_post = pl.pallas_call(_tc_post_body, out_shape=_f32(N, L_OUT))


def _pad_w(w, rows, cols):
    return jnp.pad(w, ((0, rows - w.shape[0]), (0, cols - w.shape[1])))


def _pad_v(v):
    return jnp.pad(v, (0, HP - v.shape[0]))[None, :]


def kernel(x, edge_index, W1_rel, W1_root, b1, g1, be1,
           W2_rel, W2_root, b2, g2, be2,
           W3_rel, W3_root, b3, lin_W, lin_b):
    x_pad = jnp.pad(x, ((0, N_PAD - N), (0, 0)))

    npad = E_PAD - E
    src2 = jnp.concatenate(
        [edge_index[0], jnp.zeros((npad,), jnp.int32)]).reshape(NB_TOT, EB)
    dst2 = jnp.concatenate(
        [edge_index[1], jnp.full((npad,), TRASH, jnp.int32)]).reshape(NB_TOT, EB)

    zeros = jnp.zeros((N_PAD, HP), jnp.float32)

    w1r = _pad_w(W1_rel, HP, F)
    w1o = _pad_w(W1_root, HP, F)
    w2r = _pad_w(W2_rel, HP, HP)
    w2o = _pad_w(W2_root, HP, HP)
    w3r = _pad_w(W3_rel, HP, HP)
    w3o = _pad_w(W3_root, HP, HP)
    lw = jnp.zeros((16, 3 * HP), jnp.float32)
    for i in range(3):
        lw = lw.at[:L_OUT, i * HP:i * HP + H].set(lin_W[:, i * H:(i + 1) * H])
    lb = jnp.pad(lin_b, (0, 16 - L_OUT))[None, :]

    sc = _sc_scatter()
    p1, r1 = _tc_pre(x_pad, w1r, w1o)
    acc1 = sc(p1, src2, dst2, zeros)
    h1, p2, r2 = _tc_mid(acc1, r1, _pad_v(b1), _pad_v(g1), _pad_v(be1),
                         w2r, w2o)
    acc2 = sc(p2, src2, dst2, zeros)
    h2, p3, r3 = _tc_mid(acc2, r2, _pad_v(b2), _pad_v(g2), _pad_v(be2),
                         w3r, w3o)
    acc3 = sc(p3, src2, dst2, zeros)
    out = _tc_post(acc3, r3, _pad_v(b3), h1, h2, lw, lb)
    return out


# speedup vs baseline: 16.3785x; 1.8283x over previous
import functools

import jax
import jax.numpy as jnp
from jax import lax
from jax.experimental import pallas as pl
from jax.experimental.pallas import tpu as pltpu
from jax.experimental.pallas import tpu_sc as plsc

N = 10000
E = 320000
F = 128
H = 20
L_OUT = 10

HP = 32
N_PAD = 10112
TRASH = N

NC = 2
NS = 16
NW = NC * NS
EB = 128
NB_W = 80
NB_TOT = NW * NB_W
E_PAD = NB_TOT * EB
ROWS_T = N_PAD // NS


def _sc_scatter_body(p_hbm, src_hbm, dst_hbm, zero_hbm, out_hbm,
                     src_v, dst_v, rows0, rows1, ptab, acc, sem0, sem1):
    c = lax.axis_index("c")
    s = lax.axis_index("s")
    wid = c * NS + s
    row0 = pl.multiple_of(s * ROWS_T, 8)

    pltpu.sync_copy(p_hbm.at[pl.ds(row0, ROWS_T)],
                    ptab.at[pl.ds(row0, ROWS_T)])
    pltpu.sync_copy(zero_hbm.at[pl.ds(row0, ROWS_T)],
                    acc.at[pl.ds(row0, ROWS_T)])
    pltpu.sync_copy(src_hbm.at[pl.ds(wid * NB_W, NB_W)], src_v)
    pltpu.sync_copy(dst_hbm.at[pl.ds(wid * NB_W, NB_W)], dst_v)
    plsc.subcore_barrier()

    pltpu.async_copy(ptab.at[src_v.at[0]], rows0, sem0)

    def step(t, carry):
        j0 = 2 * t
        j1 = 2 * t + 1
        j2 = jnp.minimum(2 * t + 2, NB_W - 1)
        pltpu.async_copy(ptab.at[src_v.at[j1]], rows1, sem1)
        pltpu.make_async_copy(ptab.at[src_v.at[j0]], rows0, sem0).wait()
        pltpu.sync_copy(rows0, acc.at[dst_v.at[j0]], add=True)
        pltpu.async_copy(ptab.at[src_v.at[j2]], rows0, sem0)
        pltpu.make_async_copy(ptab.at[src_v.at[j1]], rows1, sem1).wait()
        pltpu.sync_copy(rows1, acc.at[dst_v.at[j1]], add=True)
        return carry

    lax.fori_loop(0, NB_W // 2, step, 0)
    pltpu.make_async_copy(ptab.at[src_v.at[NB_W - 1]], rows0, sem0).wait()
    plsc.subcore_barrier()
    pltpu.sync_copy(acc.at[pl.ds(row0, ROWS_T)],
                    out_hbm.at[c].at[pl.ds(row0, ROWS_T)])


@functools.cache
def _sc_scatter():
    return pl.kernel(
        _sc_scatter_body,
        out_type=jax.ShapeDtypeStruct((NC, N_PAD, HP), jnp.float32),
        mesh=plsc.VectorSubcoreMesh(core_axis_name="c", subcore_axis_name="s",
                                    num_cores=NC, num_subcores=NS),
        compiler_params=pltpu.CompilerParams(use_tc_tiling_on_sc=False),
        scratch_types=[
            pltpu.VMEM((NB_W, EB), jnp.int32),
            pltpu.VMEM((NB_W, EB), jnp.int32),
            pltpu.VMEM((EB, HP), jnp.float32),
            pltpu.VMEM((EB, HP), jnp.float32),
            pltpu.VMEM_SHARED((N_PAD, HP), jnp.float32),
            pltpu.VMEM_SHARED((N_PAD, HP), jnp.float32),
            pltpu.SemaphoreType.DMA,
            pltpu.SemaphoreType.DMA,
        ],
    )


def _dot_t(a, b):
    return lax.dot_general(a, b, (((1,), (1,)), ((), ())),
                           precision=lax.Precision.HIGHEST,
                           preferred_element_type=jnp.float32)


def _tc_pre_body(x_ref, wrel_ref, wroot_ref, p_ref, r_ref):
    x = x_ref[...]
    p_ref[...] = _dot_t(x, wrel_ref[...])
    r_ref[...] = _dot_t(x, wroot_ref[...])


def _tc_mid_body(acc_ref, r_ref, b_ref, g_ref, be_ref, wrel_ref, wroot_ref,
                 h_ref, p_ref, rn_ref):
    s = acc_ref[0] + acc_ref[1] + r_ref[...] + b_ref[...]
    t = jnp.maximum(s, 0.0)
    tv = t[:N]
    mu = jnp.mean(tv, axis=0, keepdims=True)
    var = jnp.mean(tv * tv, axis=0, keepdims=True) - mu * mu
    h = (t - mu) * lax.rsqrt(var + 1e-5) * g_ref[...] + be_ref[...]
    h_ref[...] = h
    p_ref[...] = _dot_t(h, wrel_ref[...])
    rn_ref[...] = _dot_t(h, wroot_ref[...])


def _tc_post_body(acc_ref, r_ref, b_ref, h1_ref, h2_ref, lw_ref, lb_ref,
                  out_ref):
    s = acc_ref[0] + acc_ref[1] + r_ref[...] + b_ref[...]
    h3 = jnp.maximum(s, 0.0)
    cat = jnp.concatenate([h1_ref[...], h2_ref[...], h3], axis=1)
    y = _dot_t(cat, lw_ref[...]) + lb_ref[...]
    out_ref[...] = y[:N, :L_OUT]


_f32 = lambda *shape: jax.ShapeDtypeStruct(shape, jnp.float32)

_tc_pre = pl.pallas_call(
    _tc_pre_body, out_shape=(_f32(N_PAD, HP), _f32(N_PAD, HP)))

_tc_mid = pl.pallas_call(
    _tc_mid_body,
    out_shape=(_f32(N_PAD, HP), _f32(N_PAD, HP), _f32(N_PAD, HP)))

_tc_post = pl.pallas_call(_tc_post_body, out_shape=_f32(N, L_OUT))


def _pad_w(w, rows, cols):
    return jnp.pad(w, ((0, rows - w.shape[0]), (0, cols - w.shape[1])))


def _pad_v(v):
    return jnp.pad(v, (0, HP - v.shape[0]))[None, :]


def kernel(x, edge_index, W1_rel, W1_root, b1, g1, be1,
           W2_rel, W2_root, b2, g2, be2,
           W3_rel, W3_root, b3, lin_W, lin_b):
    x_pad = jnp.pad(x, ((0, N_PAD - N), (0, 0)))

    npad = E_PAD - E
    src2 = jnp.concatenate(
        [edge_index[0], jnp.zeros((npad,), jnp.int32)]).reshape(NB_TOT, EB)
    dst2 = jnp.concatenate(
        [edge_index[1], jnp.full((npad,), TRASH, jnp.int32)]).reshape(NB_TOT, EB)

    zeros = jnp.zeros((N_PAD, HP), jnp.float32)

    w1r = _pad_w(W1_rel, HP, F)
    w1o = _pad_w(W1_root, HP, F)
    w2r = _pad_w(W2_rel, HP, HP)
    w2o = _pad_w(W2_root, HP, HP)
    w3r = _pad_w(W3_rel, HP, HP)
    w3o = _pad_w(W3_root, HP, HP)
    lw = jnp.zeros((16, 3 * HP), jnp.float32)
    for i in range(3):
        lw = lw.at[:L_OUT, i * HP:i * HP + H].set(lin_W[:, i * H:(i + 1) * H])
    lb = jnp.pad(lin_b, (0, 16 - L_OUT))[None, :]

    sc = _sc_scatter()
    p1, r1 = _tc_pre(x_pad, w1r, w1o)
    acc1 = sc(p1, src2, dst2, zeros)
    h1, p2, r2 = _tc_mid(acc1, r1, _pad_v(b1), _pad_v(g1), _pad_v(be1),
                         w2r, w2o)
    acc2 = sc(p2, src2, dst2, zeros)
    h2, p3, r3 = _tc_mid(acc2, r2, _pad_v(b2), _pad_v(g2), _pad_v(be2),
                         w3r, w3o)
    acc3 = sc(p3, src2, dst2, zeros)
    out = _tc_post(acc3, r3, _pad_v(b3), h1, h2, lw, lb)
    return out


# speedup vs baseline: 17.7270x; 1.0823x over previous
import functools

import jax
import jax.numpy as jnp
from jax import lax
from jax.experimental import pallas as pl
from jax.experimental.pallas import tpu as pltpu
from jax.experimental.pallas import tpu_sc as plsc

N = 10000
E = 320000
F = 128
H = 20
L_OUT = 10

HP = 24
N_PAD = 10112
TRASH = N

NC = 2
NS = 16
NW = NC * NS
EB = 128
NB_W = 80
NB_TOT = NW * NB_W
E_PAD = NB_TOT * EB
ROWS_T = N_PAD // NS


def _sc_scatter_body(p_hbm, src_hbm, dst_hbm, zero_hbm, out_hbm,
                     src_v, dst_v, rows0, rows1, ptab, acc, sem0, sem1):
    c = lax.axis_index("c")
    s = lax.axis_index("s")
    wid = c * NS + s
    row0 = pl.multiple_of(s * ROWS_T, 8)

    pltpu.sync_copy(p_hbm.at[pl.ds(row0, ROWS_T)],
                    ptab.at[pl.ds(row0, ROWS_T)])
    pltpu.sync_copy(zero_hbm.at[pl.ds(row0, ROWS_T)],
                    acc.at[pl.ds(row0, ROWS_T)])
    pltpu.sync_copy(src_hbm.at[pl.ds(wid * NB_W, NB_W)], src_v)
    pltpu.sync_copy(dst_hbm.at[pl.ds(wid * NB_W, NB_W)], dst_v)
    plsc.subcore_barrier()

    pltpu.async_copy(ptab.at[src_v.at[0]], rows0, sem0)

    def step(t, carry):
        j0 = 2 * t
        j1 = 2 * t + 1
        j2 = jnp.minimum(2 * t + 2, NB_W - 1)
        pltpu.async_copy(ptab.at[src_v.at[j1]], rows1, sem1)
        pltpu.make_async_copy(ptab.at[src_v.at[j0]], rows0, sem0).wait()
        pltpu.sync_copy(rows0, acc.at[dst_v.at[j0]], add=True)
        pltpu.async_copy(ptab.at[src_v.at[j2]], rows0, sem0)
        pltpu.make_async_copy(ptab.at[src_v.at[j1]], rows1, sem1).wait()
        pltpu.sync_copy(rows1, acc.at[dst_v.at[j1]], add=True)
        return carry

    lax.fori_loop(0, NB_W // 2, step, 0)
    pltpu.make_async_copy(ptab.at[src_v.at[NB_W - 1]], rows0, sem0).wait()
    plsc.subcore_barrier()
    pltpu.sync_copy(acc.at[pl.ds(row0, ROWS_T)],
                    out_hbm.at[c].at[pl.ds(row0, ROWS_T)])


@functools.cache
def _sc_scatter():
    return pl.kernel(
        _sc_scatter_body,
        out_type=jax.ShapeDtypeStruct((NC, N_PAD, HP), jnp.float32),
        mesh=plsc.VectorSubcoreMesh(core_axis_name="c", subcore_axis_name="s",
                                    num_cores=NC, num_subcores=NS),
        compiler_params=pltpu.CompilerParams(use_tc_tiling_on_sc=False),
        scratch_types=[
            pltpu.VMEM((NB_W, EB), jnp.int32),
            pltpu.VMEM((NB_W, EB), jnp.int32),
            pltpu.VMEM((EB, HP), jnp.float32),
            pltpu.VMEM((EB, HP), jnp.float32),
            pltpu.VMEM_SHARED((N_PAD, HP), jnp.float32),
            pltpu.VMEM_SHARED((N_PAD, HP), jnp.float32),
            pltpu.SemaphoreType.DMA,
            pltpu.SemaphoreType.DMA,
        ],
    )


def _dot_t(a, b):
    return lax.dot_general(a, b, (((1,), (1,)), ((), ())),
                           precision=lax.Precision.HIGHEST,
                           preferred_element_type=jnp.float32)


def _tc_pre_body(x_ref, wrel_ref, wroot_ref, p_ref, r_ref):
    x = x_ref[...]
    p_ref[...] = _dot_t(x, wrel_ref[...])
    r_ref[...] = _dot_t(x, wroot_ref[...])


def _tc_mid_body(acc_ref, r_ref, b_ref, g_ref, be_ref, wrel_ref, wroot_ref,
                 h_ref, p_ref, rn_ref):
    s = acc_ref[0] + acc_ref[1] + r_ref[...] + b_ref[...]
    t = jnp.maximum(s, 0.0)
    tv = t[:N]
    mu = jnp.mean(tv, axis=0, keepdims=True)
    var = jnp.mean(tv * tv, axis=0, keepdims=True) - mu * mu
    h = (t - mu) * lax.rsqrt(var + 1e-5) * g_ref[...] + be_ref[...]
    h_ref[...] = h
    p_ref[...] = _dot_t(h, wrel_ref[...])
    rn_ref[...] = _dot_t(h, wroot_ref[...])


def _tc_post_body(acc_ref, r_ref, b_ref, h1_ref, h2_ref, lw_ref, lb_ref,
                  out_ref):
    s = acc_ref[0] + acc_ref[1] + r_ref[...] + b_ref[...]
    h3 = jnp.maximum(s, 0.0)
    cat = jnp.concatenate([h1_ref[...], h2_ref[...], h3], axis=1)
    y = _dot_t(cat, lw_ref[...]) + lb_ref[...]
    out_ref[...] = y[:N, :L_OUT]


_f32 = lambda *shape: jax.ShapeDtypeStruct(shape, jnp.float32)

_tc_pre = pl.pallas_call(
    _tc_pre_body, out_shape=(_f32(N_PAD, HP), _f32(N_PAD, HP)))

_tc_mid = pl.pallas_call(
    _tc_mid_body,
    out_shape=(_f32(N_PAD, HP), _f32(N_PAD, HP), _f32(N_PAD, HP)))

_tc_post = pl.pallas_call(_tc_post_body, out_shape=_f32(N, L_OUT))


def _pad_w(w, rows, cols):
    return jnp.pad(w, ((0, rows - w.shape[0]), (0, cols - w.shape[1])))


def _pad_v(v):
    return jnp.pad(v, (0, HP - v.shape[0]))[None, :]


def kernel(x, edge_index, W1_rel, W1_root, b1, g1, be1,
           W2_rel, W2_root, b2, g2, be2,
           W3_rel, W3_root, b3, lin_W, lin_b):
    x_pad = jnp.pad(x, ((0, N_PAD - N), (0, 0)))

    npad = E_PAD - E
    src2 = jnp.concatenate(
        [edge_index[0], jnp.zeros((npad,), jnp.int32)]).reshape(NB_TOT, EB)
    dst2 = jnp.concatenate(
        [edge_index[1], jnp.full((npad,), TRASH, jnp.int32)]).reshape(NB_TOT, EB)

    zeros = jnp.zeros((N_PAD, HP), jnp.float32)

    w1r = _pad_w(W1_rel, HP, F)
    w1o = _pad_w(W1_root, HP, F)
    w2r = _pad_w(W2_rel, HP, HP)
    w2o = _pad_w(W2_root, HP, HP)
    w3r = _pad_w(W3_rel, HP, HP)
    w3o = _pad_w(W3_root, HP, HP)
    lw = jnp.zeros((16, 3 * HP), jnp.float32)
    for i in range(3):
        lw = lw.at[:L_OUT, i * HP:i * HP + H].set(lin_W[:, i * H:(i + 1) * H])
    lb = jnp.pad(lin_b, (0, 16 - L_OUT))[None, :]

    sc = _sc_scatter()
    p1, r1 = _tc_pre(x_pad, w1r, w1o)
    acc1 = sc(p1, src2, dst2, zeros)
    h1, p2, r2 = _tc_mid(acc1, r1, _pad_v(b1), _pad_v(g1), _pad_v(be1),
                         w2r, w2o)
    acc2 = sc(p2, src2, dst2, zeros)
    h2, p3, r3 = _tc_mid(acc2, r2, _pad_v(b2), _pad_v(g2), _pad_v(be2),
                         w3r, w3o)
    acc3 = sc(p3, src2, dst2, zeros)
    out = _tc_post(acc3, r3, _pad_v(b3), h1, h2, lw, lb)
    return out


# speedup vs baseline: 21.6818x; 1.2231x over previous
import functools

import jax
import jax.numpy as jnp
from jax import lax
from jax.experimental import pallas as pl
from jax.experimental.pallas import tpu as pltpu
from jax.experimental.pallas import tpu_sc as plsc

N = 10000
E = 320000
F = 128
H = 20
L_OUT = 10

HP = 32
N_PAD = 10112
NPK = N_PAD // 4
NVK = N // 4
TRASH = N

NC = 2
NS = 16
NW = NC * NS
EB = 128
NB_W = 80
NB_TOT = NW * NB_W
E_PAD = NB_TOT * EB
ROWS_T = N_PAD // NS


def _sc_scatter_body(p_hbm, src_hbm, dst_hbm, zero_hbm, out_hbm,
                     src_v, dst_v, rows0, rows1, ptab, acc, sem0, sem1):
    c = lax.axis_index("c")
    s = lax.axis_index("s")
    wid = c * NS + s
    row0 = pl.multiple_of(s * ROWS_T, 8)

    pltpu.sync_copy(p_hbm.at[pl.ds(row0, ROWS_T)],
                    ptab.at[pl.ds(row0, ROWS_T)])
    pltpu.sync_copy(zero_hbm.at[pl.ds(row0, ROWS_T)],
                    acc.at[pl.ds(row0, ROWS_T)])
    pltpu.sync_copy(src_hbm.at[pl.ds(wid * NB_W, NB_W)], src_v)
    pltpu.sync_copy(dst_hbm.at[pl.ds(wid * NB_W, NB_W)], dst_v)
    plsc.subcore_barrier()

    pltpu.async_copy(ptab.at[src_v.at[0]], rows0, sem0)

    def step(t, carry):
        j0 = 2 * t
        j1 = 2 * t + 1
        j2 = jnp.minimum(2 * t + 2, NB_W - 1)
        pltpu.async_copy(ptab.at[src_v.at[j1]], rows1, sem1)
        pltpu.make_async_copy(ptab.at[src_v.at[j0]], rows0, sem0).wait()
        pltpu.sync_copy(rows0, acc.at[dst_v.at[j0]], add=True)
        pltpu.async_copy(ptab.at[src_v.at[j2]], rows0, sem0)
        pltpu.make_async_copy(ptab.at[src_v.at[j1]], rows1, sem1).wait()
        pltpu.sync_copy(rows1, acc.at[dst_v.at[j1]], add=True)
        return carry

    lax.fori_loop(0, NB_W // 2, step, 0)
    pltpu.make_async_copy(ptab.at[src_v.at[NB_W - 1]], rows0, sem0).wait()
    plsc.subcore_barrier()
    pltpu.sync_copy(acc.at[pl.ds(row0, ROWS_T)],
                    out_hbm.at[c].at[pl.ds(row0, ROWS_T)])


@functools.cache
def _sc_scatter():
    return pl.kernel(
        _sc_scatter_body,
        out_type=jax.ShapeDtypeStruct((NC, N_PAD, HP), jnp.float32),
        mesh=plsc.VectorSubcoreMesh(core_axis_name="c", subcore_axis_name="s",
                                    num_cores=NC, num_subcores=NS),
        compiler_params=pltpu.CompilerParams(use_tc_tiling_on_sc=False),
        scratch_types=[
            pltpu.VMEM((NB_W, EB), jnp.int32),
            pltpu.VMEM((NB_W, EB), jnp.int32),
            pltpu.VMEM((EB, HP), jnp.float32),
            pltpu.VMEM((EB, HP), jnp.float32),
            pltpu.VMEM_SHARED((N_PAD, HP), jnp.float32),
            pltpu.VMEM_SHARED((N_PAD, HP), jnp.float32),
            pltpu.SemaphoreType.DMA,
            pltpu.SemaphoreType.DMA,
        ],
    )


def _dot_t(a, b):
    return lax.dot_general(a, b, (((1,), (1,)), ((), ())),
                           precision=lax.Precision.HIGHEST,
                           preferred_element_type=jnp.float32)


def _fold4(v):
    return (v + jnp.roll(v, 32, axis=1) + jnp.roll(v, 64, axis=1)
            + jnp.roll(v, 96, axis=1))


def _tc_pre_body(x_ref, wrel_ref, wroot_ref, p_ref, r_ref):
    x = x_ref[...]
    p_ref[...] = _dot_t(x, wrel_ref[...])
    r_ref[...] = _dot_t(x, wroot_ref[...])


def _tc_mid_body(acc_ref, r_ref, b_ref, g_ref, be_ref, wrel_ref, wroot_ref,
                 h_ref, p_ref, rn_ref):
    s = acc_ref[0] + acc_ref[1] + r_ref[...] + b_ref[...]
    t = jnp.maximum(s, 0.0)
    tv = t[:NVK]
    mu = _fold4(jnp.sum(tv, axis=0, keepdims=True)) * (1.0 / N)
    m2 = _fold4(jnp.sum(tv * tv, axis=0, keepdims=True)) * (1.0 / N)
    var = m2 - mu * mu
    h = (t - mu) * lax.rsqrt(var + 1e-5) * g_ref[...] + be_ref[...]
    h_ref[...] = h
    p_ref[...] = _dot_t(h, wrel_ref[...])
    rn_ref[...] = _dot_t(h, wroot_ref[...])


def _tc_post_body(acc_ref, r_ref, b_ref, h1_ref, h2_ref, a1_ref, a2_ref,
                  a3_ref, lb_ref, out_ref):
    s = acc_ref[0] + acc_ref[1] + r_ref[...] + b_ref[...]
    h3 = jnp.maximum(s, 0.0)
    out_ref[...] = (_dot_t(h1_ref[...], a1_ref[...])
                    + _dot_t(h2_ref[...], a2_ref[...])
                    + _dot_t(h3, a3_ref[...]) + lb_ref[...])


_f32 = lambda *shape: jax.ShapeDtypeStruct(shape, jnp.float32)

_tc_pre = pl.pallas_call(
    _tc_pre_body, out_shape=(_f32(NPK, 128), _f32(NPK, 128)))

_tc_mid = pl.pallas_call(
    _tc_mid_body,
    out_shape=(_f32(NPK, 128), _f32(NPK, 128), _f32(NPK, 128)))

_tc_post = pl.pallas_call(_tc_post_body, out_shape=_f32(NPK, 128))


_EYE4 = None


def _pad_w(w, rows, cols):
    return jnp.pad(w, ((0, rows - w.shape[0]), (0, cols - w.shape[1])))


def _kron4(w):
    return jnp.kron(jnp.eye(4, dtype=w.dtype), w)


def _tile_v(v):
    return jnp.tile(jnp.pad(v, (0, HP - v.shape[0])), 4)[None, :]


def kernel(x, edge_index, W1_rel, W1_root, b1, g1, be1,
           W2_rel, W2_root, b2, g2, be2,
           W3_rel, W3_root, b3, lin_W, lin_b):
    x4 = jnp.pad(x, ((0, N_PAD - N), (0, 0))).reshape(NPK, 4 * F)

    npad = E_PAD - E
    src2 = jnp.concatenate(
        [edge_index[0], jnp.zeros((npad,), jnp.int32)]).reshape(NB_TOT, EB)
    dst2 = jnp.concatenate(
        [edge_index[1], jnp.full((npad,), TRASH, jnp.int32)]).reshape(NB_TOT, EB)

    zeros = jnp.zeros((N_PAD, HP), jnp.float32)

    w1r = _kron4(_pad_w(W1_rel, HP, F))
    w1o = _kron4(_pad_w(W1_root, HP, F))
    w2r = _kron4(_pad_w(W2_rel, HP, HP))
    w2o = _kron4(_pad_w(W2_root, HP, HP))
    w3r = _kron4(_pad_w(W3_rel, HP, HP))
    w3o = _kron4(_pad_w(W3_root, HP, HP))
    a1 = _kron4(_pad_w(lin_W[:, 0:H], HP, HP))
    a2 = _kron4(_pad_w(lin_W[:, H:2 * H], HP, HP))
    a3 = _kron4(_pad_w(lin_W[:, 2 * H:3 * H], HP, HP))
    lb = _tile_v(lin_b)

    sc = _sc_scatter()
    p1, r1 = _tc_pre(x4, w1r, w1o)
    acc1 = sc(p1.reshape(N_PAD, HP), src2, dst2, zeros)
    h1, p2, r2 = _tc_mid(acc1.reshape(NC, NPK, 128), r1,
                         _tile_v(b1), _tile_v(g1), _tile_v(be1), w2r, w2o)
    acc2 = sc(p2.reshape(N_PAD, HP), src2, dst2, zeros)
    h2, p3, r3 = _tc_mid(acc2.reshape(NC, NPK, 128), r2,
                         _tile_v(b2), _tile_v(g2), _tile_v(be2), w3r, w3o)
    acc3 = sc(p3.reshape(N_PAD, HP), src2, dst2, zeros)
    outp = _tc_post(acc3.reshape(NC, NPK, 128), r3, _tile_v(b3),
                    h1, h2, a1, a2, a3, lb)
    return outp.reshape(N_PAD, HP)[:N, :L_OUT]


# speedup vs baseline: 23.9686x; 1.1055x over previous
import functools

import jax
import jax.numpy as jnp
from jax import lax
from jax.experimental import pallas as pl
from jax.experimental.pallas import tpu as pltpu
from jax.experimental.pallas import tpu_sc as plsc

N = 10000
E = 320000
F = 128
H = 20
L_OUT = 10

HP = 32
N_PAD = 10112
NPK = N_PAD // 4
NVK = N // 4

NC = 2
NS = 16
NW = NC * NS
EB = 128
NB_TOT = E // EB
NB_W = NB_TOT // NW
NB_X = NB_TOT - NB_W * NW
ROWS_T = N_PAD // NS


def _sc_scatter_body(p_hbm, edge_hbm, zero_hbm, out_hbm,
                     src_v, dst_v, rows0, rows1, ptab, acc, sem0, sem1):
    c = lax.axis_index("c")
    s = lax.axis_index("s")
    wid = c * NS + s
    row0 = pl.multiple_of(s * ROWS_T, 8)

    pltpu.sync_copy(p_hbm.at[pl.ds(row0, ROWS_T)],
                    ptab.at[pl.ds(row0, ROWS_T)])
    pltpu.sync_copy(zero_hbm.at[pl.ds(row0, ROWS_T)],
                    acc.at[pl.ds(row0, ROWS_T)])
    pltpu.sync_copy(edge_hbm.at[0].at[pl.ds(wid * NB_W, NB_W)],
                    src_v.at[pl.ds(0, NB_W)])
    pltpu.sync_copy(edge_hbm.at[1].at[pl.ds(wid * NB_W, NB_W)],
                    dst_v.at[pl.ds(0, NB_W)])

    @pl.when(wid < NB_X)
    def _():
        xb = NW * NB_W + wid
        pltpu.sync_copy(edge_hbm.at[0].at[pl.ds(xb, 1)],
                        src_v.at[pl.ds(NB_W, 1)])
        pltpu.sync_copy(edge_hbm.at[1].at[pl.ds(xb, 1)],
                        dst_v.at[pl.ds(NB_W, 1)])

    plsc.subcore_barrier()

    pltpu.async_copy(ptab.at[src_v.at[0]], rows0, sem0)

    def step(t, carry):
        j0 = 2 * t
        j1 = 2 * t + 1
        j2 = jnp.minimum(2 * t + 2, NB_W - 1)
        pltpu.async_copy(ptab.at[src_v.at[j1]], rows1, sem1)
        pltpu.make_async_copy(ptab.at[src_v.at[j0]], rows0, sem0).wait()
        pltpu.sync_copy(rows0, acc.at[dst_v.at[j0]], add=True)
        pltpu.async_copy(ptab.at[src_v.at[j2]], rows0, sem0)
        pltpu.make_async_copy(ptab.at[src_v.at[j1]], rows1, sem1).wait()
        pltpu.sync_copy(rows1, acc.at[dst_v.at[j1]], add=True)
        return carry

    lax.fori_loop(0, NB_W // 2, step, 0)
    pltpu.make_async_copy(ptab.at[src_v.at[NB_W - 1]], rows0, sem0).wait()

    @pl.when(wid < NB_X)
    def _():
        pltpu.async_copy(ptab.at[src_v.at[NB_W]], rows0, sem0)
        pltpu.make_async_copy(ptab.at[src_v.at[NB_W]], rows0, sem0).wait()
        pltpu.sync_copy(rows0, acc.at[dst_v.at[NB_W]], add=True)

    plsc.subcore_barrier()
    pltpu.sync_copy(acc.at[pl.ds(row0, ROWS_T)],
                    out_hbm.at[c].at[pl.ds(row0, ROWS_T)])


@functools.cache
def _sc_scatter():
    return pl.kernel(
        _sc_scatter_body,
        out_type=jax.ShapeDtypeStruct((NC, N_PAD, HP), jnp.float32),
        mesh=plsc.VectorSubcoreMesh(core_axis_name="c", subcore_axis_name="s",
                                    num_cores=NC, num_subcores=NS),
        compiler_params=pltpu.CompilerParams(use_tc_tiling_on_sc=False),
        scratch_types=[
            pltpu.VMEM((NB_W + 1, EB), jnp.int32),
            pltpu.VMEM((NB_W + 1, EB), jnp.int32),
            pltpu.VMEM((EB, HP), jnp.float32),
            pltpu.VMEM((EB, HP), jnp.float32),
            pltpu.VMEM_SHARED((N_PAD, HP), jnp.float32),
            pltpu.VMEM_SHARED((N_PAD, HP), jnp.float32),
            pltpu.SemaphoreType.DMA,
            pltpu.SemaphoreType.DMA,
        ],
    )


def _dot_t(a, b):
    return lax.dot_general(a, b, (((1,), (1,)), ((), ())),
                           precision=lax.Precision.HIGHEST,
                           preferred_element_type=jnp.float32)


def _fold4(v):
    return (v + jnp.roll(v, 32, axis=1) + jnp.roll(v, 64, axis=1)
            + jnp.roll(v, 96, axis=1))


def _tc_pre_body(x_ref, wrel_ref, wroot_ref, p_ref, r_ref):
    x = x_ref[...]
    p_ref[:NVK] = _dot_t(x, wrel_ref[...])
    r_ref[:NVK] = _dot_t(x, wroot_ref[...])


def _tc_mid_body(acc_ref, r_ref, b_ref, g_ref, be_ref, wrel_ref, wroot_ref,
                 h_ref, p_ref, rn_ref):
    s = acc_ref[0] + acc_ref[1] + r_ref[...] + b_ref[...]
    t = jnp.maximum(s, 0.0)
    tv = t[:NVK]
    mu = _fold4(jnp.sum(tv, axis=0, keepdims=True)) * (1.0 / N)
    m2 = _fold4(jnp.sum(tv * tv, axis=0, keepdims=True)) * (1.0 / N)
    var = m2 - mu * mu
    h = (t - mu) * lax.rsqrt(var + 1e-5) * g_ref[...] + be_ref[...]
    h_ref[...] = h
    p_ref[...] = _dot_t(h, wrel_ref[...])
    rn_ref[...] = _dot_t(h, wroot_ref[...])


def _tc_post_body(acc_ref, r_ref, b_ref, h1_ref, h2_ref, a1_ref, a2_ref,
                  a3_ref, lb_ref, out_ref):
    s = acc_ref[0] + acc_ref[1] + r_ref[...] + b_ref[...]
    h3 = jnp.maximum(s, 0.0)
    out_ref[...] = (_dot_t(h1_ref[...], a1_ref[...])
                    + _dot_t(h2_ref[...], a2_ref[...])
                    + _dot_t(h3, a3_ref[...]) + lb_ref[...])


_f32 = lambda *shape: jax.ShapeDtypeStruct(shape, jnp.float32)

_tc_pre = pl.pallas_call(
    _tc_pre_body, out_shape=(_f32(NPK, 128), _f32(NPK, 128)))

_tc_mid = pl.pallas_call(
    _tc_mid_body,
    out_shape=(_f32(NPK, 128), _f32(NPK, 128), _f32(NPK, 128)))

_tc_post = pl.pallas_call(_tc_post_body, out_shape=_f32(NPK, 128))


_EYE4 = None


def _pad_w(w, rows, cols):
    return jnp.pad(w, ((0, rows - w.shape[0]), (0, cols - w.shape[1])))


def _kron4(w):
    return jnp.kron(jnp.eye(4, dtype=w.dtype), w)


def _tile_v(v):
    return jnp.tile(jnp.pad(v, (0, HP - v.shape[0])), 4)[None, :]


def kernel(x, edge_index, W1_rel, W1_root, b1, g1, be1,
           W2_rel, W2_root, b2, g2, be2,
           W3_rel, W3_root, b3, lin_W, lin_b):
    x4 = x.reshape(NVK, 4 * F)
    ei3 = edge_index.reshape(2, NB_TOT, EB)
    zeros = jnp.zeros((N_PAD, HP), jnp.float32)

    w1r = _kron4(_pad_w(W1_rel, HP, F))
    w1o = _kron4(_pad_w(W1_root, HP, F))
    w2r = _kron4(_pad_w(W2_rel, HP, HP))
    w2o = _kron4(_pad_w(W2_root, HP, HP))
    w3r = _kron4(_pad_w(W3_rel, HP, HP))
    w3o = _kron4(_pad_w(W3_root, HP, HP))
    a1 = _kron4(_pad_w(lin_W[:, 0:H], HP, HP))
    a2 = _kron4(_pad_w(lin_W[:, H:2 * H], HP, HP))
    a3 = _kron4(_pad_w(lin_W[:, 2 * H:3 * H], HP, HP))
    lb = _tile_v(lin_b)

    sc = _sc_scatter()
    p1, r1 = _tc_pre(x4, w1r, w1o)
    acc1 = sc(p1.reshape(N_PAD, HP), ei3, zeros)
    h1, p2, r2 = _tc_mid(acc1.reshape(NC, NPK, 128), r1,
                         _tile_v(b1), _tile_v(g1), _tile_v(be1), w2r, w2o)
    acc2 = sc(p2.reshape(N_PAD, HP), ei3, zeros)
    h2, p3, r3 = _tc_mid(acc2.reshape(NC, NPK, 128), r2,
                         _tile_v(b2), _tile_v(g2), _tile_v(be2), w3r, w3o)
    acc3 = sc(p3.reshape(N_PAD, HP), ei3, zeros)
    outp = _tc_post(acc3.reshape(NC, NPK, 128), r3, _tile_v(b3),
                    h1, h2, a1, a2, a3, lb)
    return outp.reshape(N_PAD, HP)[:N, :L_OUT]


# speedup vs baseline: 24.4279x; 1.0192x over previous
import functools

import jax
import jax.numpy as jnp
from jax import lax
from jax.experimental import pallas as pl
from jax.experimental.pallas import tpu as pltpu
from jax.experimental.pallas import tpu_sc as plsc

N = 10000
E = 320000
F = 128
H = 20
L_OUT = 10

HP = 32
N_PAD = 10112
NPK = N_PAD // 4
NVK = N // 4

NC = 2
NS = 16
NW = NC * NS
EB = 128
NB_TOT = E // EB
NB_W = NB_TOT // NW
NB_X = NB_TOT - NB_W * NW
ROWS_T = N_PAD // NS


def _sc_scatter_body(p_hbm, edge_hbm, zero_hbm, out_hbm,
                     src_v, dst_v, rows0, rows1, ptab, acc, sem0, sem1):
    c = lax.axis_index("c")
    s = lax.axis_index("s")
    wid = c * NS + s
    row0 = pl.multiple_of(s * ROWS_T, 8)

    pltpu.sync_copy(p_hbm.at[pl.ds(row0, ROWS_T)],
                    ptab.at[pl.ds(row0, ROWS_T)])
    pltpu.sync_copy(zero_hbm.at[pl.ds(row0, ROWS_T)],
                    acc.at[pl.ds(row0, ROWS_T)])
    pltpu.sync_copy(edge_hbm.at[0].at[pl.ds(wid * NB_W, NB_W)],
                    src_v.at[pl.ds(0, NB_W)])
    pltpu.sync_copy(edge_hbm.at[1].at[pl.ds(wid * NB_W, NB_W)],
                    dst_v.at[pl.ds(0, NB_W)])

    @pl.when(wid < NB_X)
    def _():
        xb = NW * NB_W + wid
        pltpu.sync_copy(edge_hbm.at[0].at[pl.ds(xb, 1)],
                        src_v.at[pl.ds(NB_W, 1)])
        pltpu.sync_copy(edge_hbm.at[1].at[pl.ds(xb, 1)],
                        dst_v.at[pl.ds(NB_W, 1)])

    plsc.subcore_barrier()

    pltpu.async_copy(ptab.at[src_v.at[0]], rows0, sem0)

    def step(t, carry):
        j0 = 2 * t
        j1 = 2 * t + 1
        j2 = jnp.minimum(2 * t + 2, NB_W - 1)
        pltpu.async_copy(ptab.at[src_v.at[j1]], rows1, sem1)
        pltpu.make_async_copy(ptab.at[src_v.at[j0]], rows0, sem0).wait()
        pltpu.sync_copy(rows0, acc.at[dst_v.at[j0]], add=True)
        pltpu.async_copy(ptab.at[src_v.at[j2]], rows0, sem0)
        pltpu.make_async_copy(ptab.at[src_v.at[j1]], rows1, sem1).wait()
        pltpu.sync_copy(rows1, acc.at[dst_v.at[j1]], add=True)
        return carry

    lax.fori_loop(0, NB_W // 2, step, 0)
    pltpu.make_async_copy(ptab.at[src_v.at[NB_W - 1]], rows0, sem0).wait()

    @pl.when(wid < NB_X)
    def _():
        pltpu.async_copy(ptab.at[src_v.at[NB_W]], rows0, sem0)
        pltpu.make_async_copy(ptab.at[src_v.at[NB_W]], rows0, sem0).wait()
        pltpu.sync_copy(rows0, acc.at[dst_v.at[NB_W]], add=True)

    plsc.subcore_barrier()
    pltpu.sync_copy(acc.at[pl.ds(row0, ROWS_T)],
                    out_hbm.at[c].at[pl.ds(row0, ROWS_T)])


@functools.cache
def _sc_scatter():
    return pl.kernel(
        _sc_scatter_body,
        out_type=jax.ShapeDtypeStruct((NC, N_PAD, HP), jnp.float32),
        mesh=plsc.VectorSubcoreMesh(core_axis_name="c", subcore_axis_name="s",
                                    num_cores=NC, num_subcores=NS),
        compiler_params=pltpu.CompilerParams(use_tc_tiling_on_sc=False),
        scratch_types=[
            pltpu.VMEM((NB_W + 1, EB), jnp.int32),
            pltpu.VMEM((NB_W + 1, EB), jnp.int32),
            pltpu.VMEM((EB, HP), jnp.float32),
            pltpu.VMEM((EB, HP), jnp.float32),
            pltpu.VMEM_SHARED((N_PAD, HP), jnp.float32),
            pltpu.VMEM_SHARED((N_PAD, HP), jnp.float32),
            pltpu.SemaphoreType.DMA,
            pltpu.SemaphoreType.DMA,
        ],
    )


def _dot_t(a, b):
    return lax.dot_general(a, b, (((1,), (1,)), ((), ())),
                           precision=lax.Precision.HIGHEST,
                           preferred_element_type=jnp.float32)


def _fold4(v):
    return (v + jnp.roll(v, 32, axis=1) + jnp.roll(v, 64, axis=1)
            + jnp.roll(v, 96, axis=1))


def _tc_pre_body(x_ref, w_ref, p_ref, r_ref):
    y = _dot_t(x_ref[...], w_ref[...])
    p_ref[:NVK] = y[:, :128]
    r_ref[:NVK] = y[:, 128:]


def _tc_mid_body(acc_ref, r_ref, b_ref, g_ref, be_ref, wrel_ref, wroot_ref,
                 h_ref, p_ref, rn_ref):
    s = acc_ref[0] + acc_ref[1] + r_ref[...] + b_ref[...]
    t = jnp.maximum(s, 0.0)
    tv = t[:NVK]
    mu = _fold4(jnp.sum(tv, axis=0, keepdims=True)) * (1.0 / N)
    m2 = _fold4(jnp.sum(tv * tv, axis=0, keepdims=True)) * (1.0 / N)
    var = m2 - mu * mu
    h = (t - mu) * lax.rsqrt(var + 1e-5) * g_ref[...] + be_ref[...]
    h_ref[...] = h
    p_ref[...] = _dot_t(h, wrel_ref[...])
    rn_ref[...] = _dot_t(h, wroot_ref[...])


def _tc_post_body(acc_ref, r_ref, b_ref, h1_ref, h2_ref, a1_ref, a2_ref,
                  a3_ref, lb_ref, out_ref):
    s = acc_ref[0] + acc_ref[1] + r_ref[...] + b_ref[...]
    h3 = jnp.maximum(s, 0.0)
    out_ref[...] = (_dot_t(h1_ref[...], a1_ref[...])
                    + _dot_t(h2_ref[...], a2_ref[...])
                    + _dot_t(h3, a3_ref[...]) + lb_ref[...])


_f32 = lambda *shape: jax.ShapeDtypeStruct(shape, jnp.float32)

_tc_pre = pl.pallas_call(
    _tc_pre_body, out_shape=(_f32(NPK, 128), _f32(NPK, 128)))

_tc_mid = pl.pallas_call(
    _tc_mid_body,
    out_shape=(_f32(NPK, 128), _f32(NPK, 128), _f32(NPK, 128)))

_tc_post = pl.pallas_call(_tc_post_body, out_shape=_f32(NPK, 128))


_EYE4 = None


def _pad_w(w, rows, cols):
    return jnp.pad(w, ((0, rows - w.shape[0]), (0, cols - w.shape[1])))


def _kron4(w):
    return jnp.kron(jnp.eye(4, dtype=w.dtype), w)


def _tile_v(v):
    return jnp.tile(jnp.pad(v, (0, HP - v.shape[0])), 4)[None, :]


def kernel(x, edge_index, W1_rel, W1_root, b1, g1, be1,
           W2_rel, W2_root, b2, g2, be2,
           W3_rel, W3_root, b3, lin_W, lin_b):
    x4 = x.reshape(NVK, 4 * F)
    ei3 = edge_index.reshape(2, NB_TOT, EB)
    zeros = jnp.zeros((N_PAD, HP), jnp.float32)

    w1r = _kron4(_pad_w(W1_rel, HP, F))
    w1o = _kron4(_pad_w(W1_root, HP, F))
    w2r = _kron4(_pad_w(W2_rel, HP, HP))
    w2o = _kron4(_pad_w(W2_root, HP, HP))
    w3r = _kron4(_pad_w(W3_rel, HP, HP))
    w3o = _kron4(_pad_w(W3_root, HP, HP))
    a1 = _kron4(_pad_w(lin_W[:, 0:H], HP, HP))
    a2 = _kron4(_pad_w(lin_W[:, H:2 * H], HP, HP))
    a3 = _kron4(_pad_w(lin_W[:, 2 * H:3 * H], HP, HP))
    lb = _tile_v(lin_b)

    sc = _sc_scatter()
    p1, r1 = _tc_pre(x4, jnp.concatenate([w1r, w1o], axis=0))
    acc1 = sc(p1.reshape(N_PAD, HP), ei3, zeros)
    h1, p2, r2 = _tc_mid(acc1.reshape(NC, NPK, 128), r1,
                         _tile_v(b1), _tile_v(g1), _tile_v(be1), w2r, w2o)
    acc2 = sc(p2.reshape(N_PAD, HP), ei3, zeros)
    h2, p3, r3 = _tc_mid(acc2.reshape(NC, NPK, 128), r2,
                         _tile_v(b2), _tile_v(g2), _tile_v(be2), w3r, w3o)
    acc3 = sc(p3.reshape(N_PAD, HP), ei3, zeros)
    outp = _tc_post(acc3.reshape(NC, NPK, 128), r3, _tile_v(b3),
                    h1, h2, a1, a2, a3, lb)
    return outp.reshape(N_PAD, HP)[:N, :L_OUT]
